# baseline (speedup 1.0000x reference)
import functools

import jax
import jax.numpy as jnp
from jax import lax
from jax.experimental import pallas as pl
from jax.experimental.pallas import tpu as pltpu
from jax.experimental.pallas import tpu_sc as plsc

_N = 2
_LQ = 5440
_C = 256
_M = 8
_L = 4
_P = 4
_LEN = 5440
_QC = 1088
_GPC = _QC // 16
_NCHUNK = _LQ // _QC

_SIZES = (64, 32, 16, 8)
_STARTS = (0, 4096, 5120, 5376)


def _stage1a(q_ref, rp_ref, woff_ref, boff_ref, wattn_ref, battn_ref,
             pxy_ref, aw_ref):
    q = q_ref[0]
    dn = (((1,), (1,)), ((), ()))
    off_t = (lax.dot_general(woff_ref[0], q, dn,
                             preferred_element_type=jnp.float32)
             + boff_ref[0])

    rp = rp_ref[0].reshape(_LQ, _L * 2).T
    rpb = jnp.broadcast_to(rp.reshape(_L, 1, 2, _LQ),
                           (_L, _P, 2, _LQ)).reshape(32, _LQ)

    ridx = lax.broadcasted_iota(jnp.int32, (32, 1), 0)
    lvl = ridx // (2 * _P)
    wl = jnp.left_shift(1, 6 - lvl).astype(jnp.float32)
    pxy_ref[0] = (rpb + off_t) * wl - 0.5

    at = (lax.dot_general(wattn_ref[0], q, dn,
                          preferred_element_type=jnp.float32)
          + battn_ref[0])
    amax = jnp.max(at, axis=0, keepdims=True)
    e = jnp.exp(at - amax)
    s = jnp.sum(e, axis=0, keepdims=True)
    aw_ref[0] = e / s


def _stage1b(x_ref, wval_ref, bval_ref, val_ref):
    x = x_ref[0]
    v = (jnp.dot(x, wval_ref[...], preferred_element_type=jnp.float32)
         + bval_ref[...])
    vt = v.T.astype(jnp.bfloat16)
    u = lax.bitcast_convert_type(vt, jnp.uint16).astype(jnp.int32)
    u = u.reshape(128, 2, _LEN)
    word = u[:, 0, :] | (u[:, 1, :] << 16)
    val_ref[0] = word.reshape(16, 8, _LEN)


def _stage3(attn_ref, wout_ref, bout_ref, out_ref):
    a = attn_ref[0].T
    out_ref[0] = (jnp.dot(a, wout_ref[...], preferred_element_type=jnp.float32)
                  + bout_ref[...])


def _sc_body(val_hbm, pxy_hbm, aw_hbm, out_hbm, val_v, pxy_v, aw_v, out_v):
    cid = lax.axis_index("c")
    sid = lax.axis_index("s")
    wid = sid * 2 + cid
    n = wid // 16
    mh = wid - n * 16
    m = mh // 2

    pltpu.sync_copy(val_hbm.at[n, mh, :], val_v)

    lenv = jnp.full((16,), _LEN, jnp.int32)
    himask = jnp.full((16,), -65536, jnp.int32)

    def chunk_body(ci, carry):
        q0 = ci * _QC
        pltpu.sync_copy(pxy_hbm.at[n, pl.ds(m * 32, 32), pl.ds(q0, _QC)],
                        pxy_v)
        pltpu.sync_copy(aw_hbm.at[n, pl.ds(m * 16, 16), pl.ds(q0, _QC)],
                        aw_v)

        @plsc.parallel_loop(0, _GPC, unroll=2)
        def group_body(g):
            gq = g * 16
            accs = [jnp.zeros((16,), jnp.float32) for _ in range(16)]
            for lp in range(16):
                wi = _SIZES[lp // _P]
                base = _STARTS[lp // _P]
                xx = pxy_v[2 * lp, pl.ds(gq, 16)]
                yy = pxy_v[2 * lp + 1, pl.ds(gq, 16)]
                aa = aw_v[lp, pl.ds(gq, 16)]

                xt = xx.astype(jnp.int32)
                xf = xt.astype(jnp.float32)
                ix0 = xt - (xf > xx).astype(jnp.int32)
                fx0 = ix0.astype(jnp.float32)
                wx1 = xx - fx0
                wx0 = 1.0 - wx1

                yt = yy.astype(jnp.int32)
                yf = yt.astype(jnp.float32)
                iy0 = yt - (yf > yy).astype(jnp.int32)
                fy0 = iy0.astype(jnp.float32)
                wy1 = yy - fy0
                wy0 = 1.0 - wy1

                zero = jnp.zeros((16,), jnp.float32)
                wx0 = jnp.where((ix0 >= 0) & (ix0 <= wi - 1), wx0, zero)
                wx1 = jnp.where((ix0 >= -1) & (ix0 <= wi - 2), wx1, zero)
                wy0 = jnp.where((iy0 >= 0) & (iy0 <= wi - 1), wy0, zero)
                wy1 = jnp.where((iy0 >= -1) & (iy0 <= wi - 2), wy1, zero)

                ix0c = jnp.clip(ix0, 0, wi - 1)
                ix1c = jnp.clip(ix0 + 1, 0, wi - 1)
                iy0c = jnp.clip(iy0, 0, wi - 1)
                iy1c = jnp.clip(iy0 + 1, 0, wi - 1)

                ax0 = wx0 * aa
                ax1 = wx1 * aa
                w00 = ax0 * wy0
                w01 = ax1 * wy0
                w10 = ax0 * wy1
                w11 = ax1 * wy1

                t0 = iy0c * wi + base
                t1 = iy1c * wi + base
                i00 = t0 + ix0c
                i01 = t0 + ix1c
                i10 = t1 + ix0c
                i11 = t1 + ix1c

                for c2 in range(8):
                    g00 = plsc.load_gather(val_v, [i00])
                    g01 = plsc.load_gather(val_v, [i01])
                    g10 = plsc.load_gather(val_v, [i10])
                    g11 = plsc.load_gather(val_v, [i11])
                    lo00 = plsc.bitcast(g00 << 16, jnp.float32)
                    lo01 = plsc.bitcast(g01 << 16, jnp.float32)
                    lo10 = plsc.bitcast(g10 << 16, jnp.float32)
                    lo11 = plsc.bitcast(g11 << 16, jnp.float32)
                    hi00 = plsc.bitcast(g00 & himask, jnp.float32)
                    hi01 = plsc.bitcast(g01 & himask, jnp.float32)
                    hi10 = plsc.bitcast(g10 & himask, jnp.float32)
                    hi11 = plsc.bitcast(g11 & himask, jnp.float32)
                    accs[2 * c2] = (accs[2 * c2] + lo00 * w00 + lo01 * w01
                                    + lo10 * w10 + lo11 * w11)
                    accs[2 * c2 + 1] = (accs[2 * c2 + 1] + hi00 * w00
                                        + hi01 * w01 + hi10 * w10
                                        + hi11 * w11)
                    if c2 < 7:
                        i00 = i00 + lenv
                        i01 = i01 + lenv
                        i10 = i10 + lenv
                        i11 = i11 + lenv
            for c in range(16):
                out_v[c, pl.ds(gq, 16)] = accs[c]

        pltpu.sync_copy(out_v, out_hbm.at[n, mh, :, pl.ds(q0, _QC)])
        return carry

    lax.fori_loop(0, _NCHUNK, chunk_body, 0)


def _sc_sample(val_t, pxy, aw):
    mesh = plsc.VectorSubcoreMesh(core_axis_name="c", subcore_axis_name="s")
    f = functools.partial(
        pl.kernel,
        out_type=jax.ShapeDtypeStruct((_N, 16, 16, _LQ), jnp.float32),
        mesh=mesh,
        scratch_types=[
            pltpu.VMEM((8 * _LEN,), jnp.int32),
            pltpu.VMEM((32, _QC), jnp.float32),
            pltpu.VMEM((16, _QC), jnp.float32),
            pltpu.VMEM((16, _QC), jnp.float32),
        ],
        compiler_params=pltpu.CompilerParams(use_tc_tiling_on_sc=False,
                                             needs_layout_passes=False,
                                             disable_bounds_checks=True),
    )(_sc_body)
    return f(val_t, pxy, aw)


def kernel(query, reference_points, input_flatten, input_spatial_shapes,
           input_level_start_index, W_off, b_off, W_attn, b_attn,
           W_val, b_val, W_out, b_out):
    del input_spatial_shapes, input_level_start_index

    pxy, aw = pl.pallas_call(
        _stage1a,
        grid=(_N, _M),
        in_specs=[
            pl.BlockSpec((1, _LQ, _C), lambda n, m: (n, 0, 0)),
            pl.BlockSpec((1, _LQ, _L, 2), lambda n, m: (n, 0, 0, 0)),
            pl.BlockSpec((1, 32, _C), lambda n, m: (m, 0, 0)),
            pl.BlockSpec((1, 32, 1), lambda n, m: (m, 0, 0)),
            pl.BlockSpec((1, 16, _C), lambda n, m: (m, 0, 0)),
            pl.BlockSpec((1, 16, 1), lambda n, m: (m, 0, 0)),
        ],
        out_specs=[
            pl.BlockSpec((1, 32, _LQ), lambda n, m: (n, m, 0)),
            pl.BlockSpec((1, 16, _LQ), lambda n, m: (n, m, 0)),
        ],
        out_shape=[
            jax.ShapeDtypeStruct((_N, 256, _LQ), jnp.float32),
            jax.ShapeDtypeStruct((_N, 128, _LQ), jnp.float32),
        ],
    )(query, reference_points,
      W_off.T.reshape(_M, 32, _C), b_off.reshape(_M, 32, 1),
      W_attn.T.reshape(_M, 16, _C), b_attn.reshape(_M, 16, 1))

    val_t = pl.pallas_call(
        _stage1b,
        grid=(_N,),
        in_specs=[
            pl.BlockSpec((1, _LQ, _C), lambda n: (n, 0, 0)),
            pl.BlockSpec((_C, _C), lambda n: (0, 0)),
            pl.BlockSpec((1, _C), lambda n: (0, 0)),
        ],
        out_specs=pl.BlockSpec((1, 16, 8, _LEN), lambda n: (n, 0, 0, 0)),
        out_shape=jax.ShapeDtypeStruct((_N, 16, 8, _LEN), jnp.int32),
    )(input_flatten, W_val, b_val.reshape(1, -1))
    val_t = val_t.reshape(_N, 16, 8 * _LEN)

    attn = _sc_sample(val_t, pxy, aw)
    attn_r = attn.reshape(_N, _C, _LQ)

    out = pl.pallas_call(
        _stage3,
        grid=(_N,),
        in_specs=[
            pl.BlockSpec((1, _C, _LQ), lambda n: (n, 0, 0)),
            pl.BlockSpec((_C, _C), lambda n: (0, 0)),
            pl.BlockSpec((1, _C), lambda n: (0, 0)),
        ],
        out_specs=pl.BlockSpec((1, _LQ, _C), lambda n: (n, 0, 0)),
        out_shape=jax.ShapeDtypeStruct((_N, _LQ, _C), jnp.float32),
    )(attn_r, W_out, b_out.reshape(1, -1))
    return out


# speedup vs baseline: 1.2454x; 1.2454x over previous
import functools

import jax
import jax.numpy as jnp
from jax import lax
from jax.experimental import pallas as pl
from jax.experimental.pallas import tpu as pltpu
from jax.experimental.pallas import tpu_sc as plsc

_N = 2
_LQ = 5440
_C = 256
_M = 8
_L = 4
_P = 4
_LEN = 5440
_QC = 1088
_GPC = _QC // 16
_NCHUNK = _LQ // _QC

_SIZES = (64, 32, 16, 8)
_STARTS = (0, 4096, 5120, 5376)


def _stage1a(q_ref, rp_ref, woff_ref, boff_ref, wattn_ref, battn_ref,
             pxy_ref, aw_ref):
    q = q_ref[0]
    dn = (((1,), (1,)), ((), ()))
    off_t = (lax.dot_general(woff_ref[0], q, dn,
                             preferred_element_type=jnp.float32)
             + boff_ref[0])

    rp = rp_ref[0].reshape(_LQ, _L * 2).T
    rpb = jnp.broadcast_to(rp.reshape(_L, 1, 2, _LQ),
                           (_L, _P, 2, _LQ)).reshape(32, _LQ)

    ridx = lax.broadcasted_iota(jnp.int32, (32, 1), 0)
    lvl = ridx // (2 * _P)
    wl = jnp.left_shift(1, 6 - lvl).astype(jnp.float32)
    pxy_ref[0] = (rpb + off_t) * wl - 0.5

    at = (lax.dot_general(wattn_ref[0], q, dn,
                          preferred_element_type=jnp.float32)
          + battn_ref[0])
    amax = jnp.max(at, axis=0, keepdims=True)
    e = jnp.exp(at - amax)
    s = jnp.sum(e, axis=0, keepdims=True)
    aw_ref[0] = e / s


def _stage1b(x_ref, wval_ref, bval_ref, val_ref):
    x = x_ref[0]
    v = (jnp.dot(x, wval_ref[...], preferred_element_type=jnp.float32)
         + bval_ref[...])
    vt = v.T.astype(jnp.bfloat16)
    u = lax.bitcast_convert_type(vt, jnp.uint16).astype(jnp.int32)
    u = u.reshape(128, 2, _LEN)
    word = u[:, 0, :] | (u[:, 1, :] << 16)
    val_ref[0] = word.reshape(16, 8, _LEN)


def _stage3(attn_ref, wout_ref, bout_ref, out_ref):
    a = attn_ref[0].T
    out_ref[0] = (jnp.dot(a, wout_ref[...], preferred_element_type=jnp.float32)
                  + bout_ref[...])


def _sc_body(val_hbm, pxy_hbm, aw_hbm, out_hbm, val_v, pxy_v, aw_v, out_v):
    cid = lax.axis_index("c")
    sid = lax.axis_index("s")
    wid = sid * 2 + cid
    n = wid // 16
    mh = wid - n * 16
    m = mh // 2

    pltpu.sync_copy(val_hbm.at[n, mh, :], val_v)

    lenv = jnp.full((16,), _LEN, jnp.int32)

    def chunk_body(ci, carry):
        q0 = ci * _QC
        pltpu.sync_copy(pxy_hbm.at[n, pl.ds(m * 32, 32), pl.ds(q0, _QC)],
                        pxy_v)
        pltpu.sync_copy(aw_hbm.at[n, pl.ds(m * 16, 16), pl.ds(q0, _QC)],
                        aw_v)

        @plsc.parallel_loop(0, _GPC)
        def group_body(g):
            gq = g * 16
            accs = [jnp.zeros((16,), jnp.float32) for _ in range(16)]
            for lp in range(16):
                wi = _SIZES[lp // _P]
                base = _STARTS[lp // _P]
                xx = pxy_v[2 * lp, pl.ds(gq, 16)]
                yy = pxy_v[2 * lp + 1, pl.ds(gq, 16)]
                aa = aw_v[lp, pl.ds(gq, 16)]

                xt = xx.astype(jnp.int32)
                xf = xt.astype(jnp.float32)
                ix0 = xt - (xf > xx).astype(jnp.int32)
                fx0 = ix0.astype(jnp.float32)
                wx1 = xx - fx0
                wx0 = 1.0 - wx1

                yt = yy.astype(jnp.int32)
                yf = yt.astype(jnp.float32)
                iy0 = yt - (yf > yy).astype(jnp.int32)
                fy0 = iy0.astype(jnp.float32)
                wy1 = yy - fy0
                wy0 = 1.0 - wy1

                zero = jnp.zeros((16,), jnp.float32)
                wx0 = jnp.where((ix0 >= 0) & (ix0 <= wi - 1), wx0, zero)
                wx1 = jnp.where((ix0 >= -1) & (ix0 <= wi - 2), wx1, zero)
                wy0 = jnp.where((iy0 >= 0) & (iy0 <= wi - 1), wy0, zero)
                wy1 = jnp.where((iy0 >= -1) & (iy0 <= wi - 2), wy1, zero)

                ix0c = jnp.clip(ix0, 0, wi - 1)
                ix1c = jnp.clip(ix0 + 1, 0, wi - 1)
                iy0c = jnp.clip(iy0, 0, wi - 1)
                iy1c = jnp.clip(iy0 + 1, 0, wi - 1)

                ax0 = wx0 * aa
                ax1 = wx1 * aa
                w00 = ax0 * wy0
                w01 = ax1 * wy0
                w10 = ax0 * wy1
                w11 = ax1 * wy1

                t0 = iy0c * wi + base
                t1 = iy1c * wi + base
                i00 = t0 + ix0c
                i01 = t0 + ix1c
                i10 = t1 + ix0c
                i11 = t1 + ix1c

                for c2 in range(8):
                    g00 = plsc.load_gather(val_v, [i00])
                    g01 = plsc.load_gather(val_v, [i01])
                    g10 = plsc.load_gather(val_v, [i10])
                    g11 = plsc.load_gather(val_v, [i11])
                    lo00 = plsc.bitcast(g00 << 16, jnp.float32)
                    lo01 = plsc.bitcast(g01 << 16, jnp.float32)
                    lo10 = plsc.bitcast(g10 << 16, jnp.float32)
                    lo11 = plsc.bitcast(g11 << 16, jnp.float32)
                    hi00 = plsc.bitcast(g00, jnp.float32)
                    hi01 = plsc.bitcast(g01, jnp.float32)
                    hi10 = plsc.bitcast(g10, jnp.float32)
                    hi11 = plsc.bitcast(g11, jnp.float32)
                    accs[2 * c2] = (accs[2 * c2] + lo00 * w00 + lo01 * w01
                                    + lo10 * w10 + lo11 * w11)
                    accs[2 * c2 + 1] = (accs[2 * c2 + 1] + hi00 * w00
                                        + hi01 * w01 + hi10 * w10
                                        + hi11 * w11)
                    if c2 < 7:
                        i00 = i00 + lenv
                        i01 = i01 + lenv
                        i10 = i10 + lenv
                        i11 = i11 + lenv
            for c in range(16):
                out_v[c, pl.ds(gq, 16)] = accs[c]

        pltpu.sync_copy(out_v, out_hbm.at[n, mh, :, pl.ds(q0, _QC)])
        return carry

    lax.fori_loop(0, _NCHUNK, chunk_body, 0)


def _sc_sample(val_t, pxy, aw):
    mesh = plsc.VectorSubcoreMesh(core_axis_name="c", subcore_axis_name="s")
    f = functools.partial(
        pl.kernel,
        out_type=jax.ShapeDtypeStruct((_N, 16, 16, _LQ), jnp.float32),
        mesh=mesh,
        scratch_types=[
            pltpu.VMEM((8 * _LEN,), jnp.int32),
            pltpu.VMEM((32, _QC), jnp.float32),
            pltpu.VMEM((16, _QC), jnp.float32),
            pltpu.VMEM((16, _QC), jnp.float32),
        ],
        compiler_params=pltpu.CompilerParams(use_tc_tiling_on_sc=False,
                                             needs_layout_passes=False,
                                             disable_bounds_checks=True),
    )(_sc_body)
    return f(val_t, pxy, aw)


def kernel(query, reference_points, input_flatten, input_spatial_shapes,
           input_level_start_index, W_off, b_off, W_attn, b_attn,
           W_val, b_val, W_out, b_out):
    del input_spatial_shapes, input_level_start_index

    pxy, aw = pl.pallas_call(
        _stage1a,
        grid=(_N, _M),
        in_specs=[
            pl.BlockSpec((1, _LQ, _C), lambda n, m: (n, 0, 0)),
            pl.BlockSpec((1, _LQ, _L, 2), lambda n, m: (n, 0, 0, 0)),
            pl.BlockSpec((1, 32, _C), lambda n, m: (m, 0, 0)),
            pl.BlockSpec((1, 32, 1), lambda n, m: (m, 0, 0)),
            pl.BlockSpec((1, 16, _C), lambda n, m: (m, 0, 0)),
            pl.BlockSpec((1, 16, 1), lambda n, m: (m, 0, 0)),
        ],
        out_specs=[
            pl.BlockSpec((1, 32, _LQ), lambda n, m: (n, m, 0)),
            pl.BlockSpec((1, 16, _LQ), lambda n, m: (n, m, 0)),
        ],
        out_shape=[
            jax.ShapeDtypeStruct((_N, 256, _LQ), jnp.float32),
            jax.ShapeDtypeStruct((_N, 128, _LQ), jnp.float32),
        ],
    )(query, reference_points,
      W_off.T.reshape(_M, 32, _C), b_off.reshape(_M, 32, 1),
      W_attn.T.reshape(_M, 16, _C), b_attn.reshape(_M, 16, 1))

    val_t = pl.pallas_call(
        _stage1b,
        grid=(_N,),
        in_specs=[
            pl.BlockSpec((1, _LQ, _C), lambda n: (n, 0, 0)),
            pl.BlockSpec((_C, _C), lambda n: (0, 0)),
            pl.BlockSpec((1, _C), lambda n: (0, 0)),
        ],
        out_specs=pl.BlockSpec((1, 16, 8, _LEN), lambda n: (n, 0, 0, 0)),
        out_shape=jax.ShapeDtypeStruct((_N, 16, 8, _LEN), jnp.int32),
    )(input_flatten, W_val, b_val.reshape(1, -1))
    val_t = val_t.reshape(_N, 16, 8 * _LEN)

    attn = _sc_sample(val_t, pxy, aw)
    attn_r = attn.reshape(_N, _C, _LQ)

    out = pl.pallas_call(
        _stage3,
        grid=(_N,),
        in_specs=[
            pl.BlockSpec((1, _C, _LQ), lambda n: (n, 0, 0)),
            pl.BlockSpec((_C, _C), lambda n: (0, 0)),
            pl.BlockSpec((1, _C), lambda n: (0, 0)),
        ],
        out_specs=pl.BlockSpec((1, _LQ, _C), lambda n: (n, 0, 0)),
        out_shape=jax.ShapeDtypeStruct((_N, _LQ, _C), jnp.float32),
    )(attn_r, W_out, b_out.reshape(1, -1))
    return out


# speedup vs baseline: 2.0007x; 1.6065x over previous
import functools

import jax
import jax.numpy as jnp
from jax import lax
from jax.experimental import pallas as pl
from jax.experimental.pallas import tpu as pltpu
from jax.experimental.pallas import tpu_sc as plsc

_N = 2
_LQ = 5440
_C = 256
_M = 8
_L = 4
_P = 4
_LEN = 5440
_QC = 544
_GPC = _QC // 16
_NCHUNK = _LQ // _QC

_SIZES = (64, 32, 16, 8)
_STARTS = (0, 4096, 5120, 5376)


def _stage1a(q_ref, rp_ref, woffx_ref, woffy_ref, boffx_ref, boffy_ref,
             wattn_ref, battn_ref, widx_ref, wwts_ref):
    q = q_ref[0]
    dn = (((1,), (1,)), ((), ()))
    offx = (lax.dot_general(woffx_ref[0], q, dn,
                            preferred_element_type=jnp.float32)
            + boffx_ref[0])
    offy = (lax.dot_general(woffy_ref[0], q, dn,
                            preferred_element_type=jnp.float32)
            + boffy_ref[0])

    rp = rp_ref[0].reshape(_LQ, _L * 2).T
    rp = rp.reshape(_L, 2, _LQ)
    rpx = jnp.broadcast_to(rp[:, None, 0, :], (_L, _P, _LQ)).reshape(16, _LQ)
    rpy = jnp.broadcast_to(rp[:, None, 1, :], (_L, _P, _LQ)).reshape(16, _LQ)

    ridx = lax.broadcasted_iota(jnp.int32, (16, 1), 0)
    lvl = ridx // _P
    wli = jnp.left_shift(1, 6 - lvl)
    wl = wli.astype(jnp.float32)

    xx = (rpx + offx) * wl - 0.5
    yy = (rpy + offy) * wl - 0.5

    at = (lax.dot_general(wattn_ref[0], q, dn,
                          preferred_element_type=jnp.float32)
          + battn_ref[0])
    amax = jnp.max(at, axis=0, keepdims=True)
    e = jnp.exp(at - amax)
    s = jnp.sum(e, axis=0, keepdims=True)
    aa = e / s

    xt = xx.astype(jnp.int32)
    xf = xt.astype(jnp.float32)
    ix0 = xt - (xf > xx).astype(jnp.int32)
    fx0 = ix0.astype(jnp.float32)
    wx1 = xx - fx0
    wx0 = 1.0 - wx1

    yt = yy.astype(jnp.int32)
    yf = yt.astype(jnp.float32)
    iy0 = yt - (yf > yy).astype(jnp.int32)
    fy0 = iy0.astype(jnp.float32)
    wy1 = yy - fy0
    wy0 = 1.0 - wy1

    wx0 = jnp.where((ix0 >= 0) & (ix0 <= wli - 1), wx0, 0.0)
    wx1 = jnp.where((ix0 >= -1) & (ix0 <= wli - 2), wx1, 0.0)
    wy0 = jnp.where((iy0 >= 0) & (iy0 <= wli - 1), wy0, 0.0)
    wy1 = jnp.where((iy0 >= -1) & (iy0 <= wli - 2), wy1, 0.0)

    ix0c = jnp.clip(ix0, 0, wli - 1)
    ix1c = jnp.clip(ix0 + 1, 0, wli - 1)
    iy0c = jnp.clip(iy0, 0, wli - 1)
    iy1c = jnp.clip(iy0 + 1, 0, wli - 1)

    ax0 = wx0 * aa
    ax1 = wx1 * aa
    w00 = ax0 * wy0
    w01 = ax1 * wy0
    w10 = ax0 * wy1
    w11 = ax1 * wy1

    base = (16384 - (16384 >> (2 * lvl))) // 3
    t0 = iy0c * wli + base
    t1 = iy1c * wli + base
    i00 = t0 + ix0c
    i01 = t0 + ix1c
    i10 = t1 + ix0c
    i11 = t1 + ix1c

    widx_ref[0, :, 0, :] = i00
    widx_ref[0, :, 1, :] = i01
    widx_ref[0, :, 2, :] = i10
    widx_ref[0, :, 3, :] = i11
    wwts_ref[0, :, 0, :] = w00
    wwts_ref[0, :, 1, :] = w01
    wwts_ref[0, :, 2, :] = w10
    wwts_ref[0, :, 3, :] = w11


def _stage1b(x_ref, wval_ref, bval_ref, val_ref):
    x = x_ref[0]
    v = (jnp.dot(x, wval_ref[...], preferred_element_type=jnp.float32)
         + bval_ref[...])
    vt = v.T.astype(jnp.bfloat16)
    u = lax.bitcast_convert_type(vt, jnp.uint16).astype(jnp.int32)
    u = u.reshape(128, 2, _LEN)
    word = u[:, 0, :] | (u[:, 1, :] << 16)
    val_ref[0] = word.reshape(16, 8, _LEN)


def _stage3(attn_ref, wout_ref, bout_ref, out_ref):
    a = attn_ref[0].T
    out_ref[0] = (jnp.dot(a, wout_ref[...], preferred_element_type=jnp.float32)
                  + bout_ref[...])


def _sc_body(val_hbm, idx_hbm, wts_hbm, out_hbm, val_v, idx_v, wts_v, out_v):
    cid = lax.axis_index("c")
    sid = lax.axis_index("s")
    wid = sid * 2 + cid
    n = wid // 16
    mh = wid - n * 16
    m = mh // 2

    pltpu.sync_copy(val_hbm.at[n, mh, :], val_v)

    lenv = jnp.full((16,), _LEN, jnp.int32)

    def chunk_body(ci, carry):
        q0 = ci * _QC
        pltpu.sync_copy(idx_hbm.at[n, pl.ds(m * 16, 16), :, pl.ds(q0, _QC)],
                        idx_v)
        pltpu.sync_copy(wts_hbm.at[n, pl.ds(m * 16, 16), :, pl.ds(q0, _QC)],
                        wts_v)

        @plsc.parallel_loop(0, _GPC)
        def group_body(g):
            gq = g * 16
            accs = [jnp.zeros((16,), jnp.float32) for _ in range(16)]
            for lp in range(16):
                i00 = idx_v[lp, 0, pl.ds(gq, 16)]
                i01 = idx_v[lp, 1, pl.ds(gq, 16)]
                i10 = idx_v[lp, 2, pl.ds(gq, 16)]
                i11 = idx_v[lp, 3, pl.ds(gq, 16)]
                w00 = wts_v[lp, 0, pl.ds(gq, 16)]
                w01 = wts_v[lp, 1, pl.ds(gq, 16)]
                w10 = wts_v[lp, 2, pl.ds(gq, 16)]
                w11 = wts_v[lp, 3, pl.ds(gq, 16)]

                for c2 in range(8):
                    g00 = plsc.load_gather(val_v, [i00])
                    g01 = plsc.load_gather(val_v, [i01])
                    g10 = plsc.load_gather(val_v, [i10])
                    g11 = plsc.load_gather(val_v, [i11])
                    lo00 = plsc.bitcast(g00 << 16, jnp.float32)
                    lo01 = plsc.bitcast(g01 << 16, jnp.float32)
                    lo10 = plsc.bitcast(g10 << 16, jnp.float32)
                    lo11 = plsc.bitcast(g11 << 16, jnp.float32)
                    hi00 = plsc.bitcast(g00, jnp.float32)
                    hi01 = plsc.bitcast(g01, jnp.float32)
                    hi10 = plsc.bitcast(g10, jnp.float32)
                    hi11 = plsc.bitcast(g11, jnp.float32)
                    accs[2 * c2] = (accs[2 * c2] + lo00 * w00 + lo01 * w01
                                    + lo10 * w10 + lo11 * w11)
                    accs[2 * c2 + 1] = (accs[2 * c2 + 1] + hi00 * w00
                                        + hi01 * w01 + hi10 * w10
                                        + hi11 * w11)
                    if c2 < 7:
                        i00 = i00 + lenv
                        i01 = i01 + lenv
                        i10 = i10 + lenv
                        i11 = i11 + lenv
            for c in range(16):
                out_v[c, pl.ds(gq, 16)] = accs[c]

        pltpu.sync_copy(out_v, out_hbm.at[n, mh, :, pl.ds(q0, _QC)])
        return carry

    lax.fori_loop(0, _NCHUNK, chunk_body, 0)


def _sc_sample(val_t, widx, wwts):
    mesh = plsc.VectorSubcoreMesh(core_axis_name="c", subcore_axis_name="s")
    f = functools.partial(
        pl.kernel,
        out_type=jax.ShapeDtypeStruct((_N, 16, 16, _LQ), jnp.float32),
        mesh=mesh,
        scratch_types=[
            pltpu.VMEM((8 * _LEN,), jnp.int32),
            pltpu.VMEM((16, 4, _QC), jnp.int32),
            pltpu.VMEM((16, 4, _QC), jnp.float32),
            pltpu.VMEM((16, _QC), jnp.float32),
        ],
        compiler_params=pltpu.CompilerParams(use_tc_tiling_on_sc=False,
                                             needs_layout_passes=False,
                                             disable_bounds_checks=True),
    )(_sc_body)
    return f(val_t, widx, wwts)


def kernel(query, reference_points, input_flatten, input_spatial_shapes,
           input_level_start_index, W_off, b_off, W_attn, b_attn,
           W_val, b_val, W_out, b_out):
    del input_spatial_shapes, input_level_start_index

    woff_t = W_off.T.reshape(_M, 16, 2, _C)
    boff_t = b_off.reshape(_M, 16, 2, 1)
    widx, wwts = pl.pallas_call(
        _stage1a,
        grid=(_N, _M),
        in_specs=[
            pl.BlockSpec((1, _LQ, _C), lambda n, m: (n, 0, 0)),
            pl.BlockSpec((1, _LQ, _L, 2), lambda n, m: (n, 0, 0, 0)),
            pl.BlockSpec((1, 16, _C), lambda n, m: (m, 0, 0)),
            pl.BlockSpec((1, 16, _C), lambda n, m: (m, 0, 0)),
            pl.BlockSpec((1, 16, 1), lambda n, m: (m, 0, 0)),
            pl.BlockSpec((1, 16, 1), lambda n, m: (m, 0, 0)),
            pl.BlockSpec((1, 16, _C), lambda n, m: (m, 0, 0)),
            pl.BlockSpec((1, 16, 1), lambda n, m: (m, 0, 0)),
        ],
        out_specs=[
            pl.BlockSpec((1, 16, 4, _LQ), lambda n, m: (n, m, 0, 0)),
            pl.BlockSpec((1, 16, 4, _LQ), lambda n, m: (n, m, 0, 0)),
        ],
        out_shape=[
            jax.ShapeDtypeStruct((_N, 128, 4, _LQ), jnp.int32),
            jax.ShapeDtypeStruct((_N, 128, 4, _LQ), jnp.float32),
        ],
        compiler_params=pltpu.CompilerParams(vmem_limit_bytes=100 * 1024 * 1024),
    )(query, reference_points,
      woff_t[:, :, 0, :], woff_t[:, :, 1, :],
      boff_t[:, :, 0, :], boff_t[:, :, 1, :],
      W_attn.T.reshape(_M, 16, _C), b_attn.reshape(_M, 16, 1))

    val_t = pl.pallas_call(
        _stage1b,
        grid=(_N,),
        in_specs=[
            pl.BlockSpec((1, _LQ, _C), lambda n: (n, 0, 0)),
            pl.BlockSpec((_C, _C), lambda n: (0, 0)),
            pl.BlockSpec((1, _C), lambda n: (0, 0)),
        ],
        out_specs=pl.BlockSpec((1, 16, 8, _LEN), lambda n: (n, 0, 0, 0)),
        out_shape=jax.ShapeDtypeStruct((_N, 16, 8, _LEN), jnp.int32),
    )(input_flatten, W_val, b_val.reshape(1, -1))
    val_t = val_t.reshape(_N, 16, 8 * _LEN)

    attn = _sc_sample(val_t, widx, wwts)
    attn_r = attn.reshape(_N, _C, _LQ)

    out = pl.pallas_call(
        _stage3,
        grid=(_N,),
        in_specs=[
            pl.BlockSpec((1, _C, _LQ), lambda n: (n, 0, 0)),
            pl.BlockSpec((_C, _C), lambda n: (0, 0)),
            pl.BlockSpec((1, _C), lambda n: (0, 0)),
        ],
        out_specs=pl.BlockSpec((1, _LQ, _C), lambda n: (n, 0, 0)),
        out_shape=jax.ShapeDtypeStruct((_N, _LQ, _C), jnp.float32),
    )(attn_r, W_out, b_out.reshape(1, -1))
    return out


# speedup vs baseline: 2.2202x; 1.1097x over previous
import functools

import jax
import jax.numpy as jnp
from jax import lax
from jax.experimental import pallas as pl
from jax.experimental.pallas import tpu as pltpu
from jax.experimental.pallas import tpu_sc as plsc

_N = 2
_LQ = 5440
_C = 256
_M = 8
_L = 4
_P = 4
_LEN = 5440
_QC = 544
_GPC = _QC // 16
_NCHUNK = _LQ // _QC

_SIZES = (64, 32, 16, 8)
_STARTS = (0, 4096, 5120, 5376)


def _stage1a(q_ref, rp_ref, woffx_ref, woffy_ref, boffx_ref, boffy_ref,
             wattn_ref, battn_ref, widx_ref, wwts_ref):
    q = q_ref[0]
    dn = (((1,), (1,)), ((), ()))
    offx = (lax.dot_general(woffx_ref[0], q, dn,
                            preferred_element_type=jnp.float32)
            + boffx_ref[0])
    offy = (lax.dot_general(woffy_ref[0], q, dn,
                            preferred_element_type=jnp.float32)
            + boffy_ref[0])

    rp = rp_ref[0].reshape(_LQ, _L * 2).T
    rp = rp.reshape(_L, 2, _LQ)
    rpx = jnp.broadcast_to(rp[:, None, 0, :], (_L, _P, _LQ)).reshape(16, _LQ)
    rpy = jnp.broadcast_to(rp[:, None, 1, :], (_L, _P, _LQ)).reshape(16, _LQ)

    ridx = lax.broadcasted_iota(jnp.int32, (16, 1), 0)
    lvl = ridx // _P
    wli = jnp.left_shift(1, 6 - lvl)
    wl = wli.astype(jnp.float32)

    xx = (rpx + offx) * wl - 0.5
    yy = (rpy + offy) * wl - 0.5

    at = (lax.dot_general(wattn_ref[0], q, dn,
                          preferred_element_type=jnp.float32)
          + battn_ref[0])
    amax = jnp.max(at, axis=0, keepdims=True)
    e = jnp.exp(at - amax)
    s = jnp.sum(e, axis=0, keepdims=True)
    aa = e / s

    xt = xx.astype(jnp.int32)
    xf = xt.astype(jnp.float32)
    ix0 = xt - (xf > xx).astype(jnp.int32)
    fx0 = ix0.astype(jnp.float32)
    wx1 = xx - fx0
    wx0 = 1.0 - wx1

    yt = yy.astype(jnp.int32)
    yf = yt.astype(jnp.float32)
    iy0 = yt - (yf > yy).astype(jnp.int32)
    fy0 = iy0.astype(jnp.float32)
    wy1 = yy - fy0
    wy0 = 1.0 - wy1

    wx0 = jnp.where((ix0 >= 0) & (ix0 <= wli - 1), wx0, 0.0)
    wx1 = jnp.where((ix0 >= -1) & (ix0 <= wli - 2), wx1, 0.0)
    wy0 = jnp.where((iy0 >= 0) & (iy0 <= wli - 1), wy0, 0.0)
    wy1 = jnp.where((iy0 >= -1) & (iy0 <= wli - 2), wy1, 0.0)

    ix0c = jnp.clip(ix0, 0, wli - 1)
    ix1c = jnp.clip(ix0 + 1, 0, wli - 1)
    iy0c = jnp.clip(iy0, 0, wli - 1)
    iy1c = jnp.clip(iy0 + 1, 0, wli - 1)

    ax0 = wx0 * aa
    ax1 = wx1 * aa
    w00 = ax0 * wy0
    w01 = ax1 * wy0
    w10 = ax0 * wy1
    w11 = ax1 * wy1

    base = (16384 - (16384 >> (2 * lvl))) // 3
    t0 = iy0c * wli + base
    t1 = iy1c * wli + base
    i00 = t0 + ix0c
    i01 = t0 + ix1c
    i10 = t1 + ix0c
    i11 = t1 + ix1c

    widx_ref[0, :, 0, :] = i00
    widx_ref[0, :, 1, :] = i01
    widx_ref[0, :, 2, :] = i10
    widx_ref[0, :, 3, :] = i11
    wwts_ref[0, :, 0, :] = w00
    wwts_ref[0, :, 1, :] = w01
    wwts_ref[0, :, 2, :] = w10
    wwts_ref[0, :, 3, :] = w11


def _stage1b(x_ref, wval_ref, bval_ref, val_ref):
    x = x_ref[0]
    v = (jnp.dot(x, wval_ref[...], preferred_element_type=jnp.float32)
         + bval_ref[...])
    vt = v.T.astype(jnp.bfloat16)
    u = lax.bitcast_convert_type(vt, jnp.uint16).astype(jnp.int32)
    u = u.reshape(128, 2, _LEN)
    word = u[:, 0, :] | (u[:, 1, :] << 16)
    val_ref[0] = word.reshape(16, 8, _LEN)


def _stage3(attn_ref, wout_ref, bout_ref, out_ref):
    a = attn_ref[0].T
    out_ref[0] = (jnp.dot(a, wout_ref[...], preferred_element_type=jnp.float32)
                  + bout_ref[...])


def _sc_body(val_hbm, idx_hbm, wts_hbm, out_hbm, val_v, idx_v, wts_v, out_v):
    cid = lax.axis_index("c")
    sid = lax.axis_index("s")
    wid = sid * 2 + cid
    n = wid // 16
    mh = wid - n * 16
    m = mh // 2

    pltpu.sync_copy(val_hbm.at[n, mh, :], val_v)

    lenv = jnp.full((16,), _LEN, jnp.int32)

    def chunk_body(ci, carry):
        q0 = ci * _QC
        pltpu.sync_copy(idx_hbm.at[n, pl.ds(m * 16, 16), :, pl.ds(q0, _QC)],
                        idx_v)
        pltpu.sync_copy(wts_hbm.at[n, pl.ds(m * 16, 16), :, pl.ds(q0, _QC)],
                        wts_v)

        @plsc.parallel_loop(0, _GPC)
        def group_body(g):
            gq = g * 16
            accs = [jnp.zeros((16,), jnp.float32) for _ in range(16)]
            for lp in range(16):
                i00 = idx_v[lp, 0, pl.ds(gq, 16)]
                i01 = idx_v[lp, 1, pl.ds(gq, 16)]
                i10 = idx_v[lp, 2, pl.ds(gq, 16)]
                i11 = idx_v[lp, 3, pl.ds(gq, 16)]
                w00 = wts_v[lp, 0, pl.ds(gq, 16)]
                w01 = wts_v[lp, 1, pl.ds(gq, 16)]
                w10 = wts_v[lp, 2, pl.ds(gq, 16)]
                w11 = wts_v[lp, 3, pl.ds(gq, 16)]
                wp00 = plsc.pack(w00, w00, format=plsc.PackFormat.INTERLEAVED)
                wp01 = plsc.pack(w01, w01, format=plsc.PackFormat.INTERLEAVED)
                wp10 = plsc.pack(w10, w10, format=plsc.PackFormat.INTERLEAVED)
                wp11 = plsc.pack(w11, w11, format=plsc.PackFormat.INTERLEAVED)

                for c2 in range(8):
                    g00 = plsc.load_gather(val_v, [i00])
                    g01 = plsc.load_gather(val_v, [i01])
                    g10 = plsc.load_gather(val_v, [i10])
                    g11 = plsc.load_gather(val_v, [i11])
                    p = (plsc.bitcast(g00, jnp.bfloat16) * wp00
                         + plsc.bitcast(g01, jnp.bfloat16) * wp01
                         + plsc.bitcast(g10, jnp.bfloat16) * wp10
                         + plsc.bitcast(g11, jnp.bfloat16) * wp11)
                    pi = plsc.bitcast(p, jnp.int32)
                    accs[2 * c2] = accs[2 * c2] + plsc.bitcast(pi << 16,
                                                               jnp.float32)
                    accs[2 * c2 + 1] = (accs[2 * c2 + 1]
                                        + plsc.bitcast(pi, jnp.float32))
                    if c2 < 7:
                        i00 = i00 + lenv
                        i01 = i01 + lenv
                        i10 = i10 + lenv
                        i11 = i11 + lenv
            for c in range(16):
                out_v[c, pl.ds(gq, 16)] = accs[c]

        pltpu.sync_copy(out_v, out_hbm.at[n, mh, :, pl.ds(q0, _QC)])
        return carry

    lax.fori_loop(0, _NCHUNK, chunk_body, 0)


def _sc_sample(val_t, widx, wwts):
    mesh = plsc.VectorSubcoreMesh(core_axis_name="c", subcore_axis_name="s")
    f = functools.partial(
        pl.kernel,
        out_type=jax.ShapeDtypeStruct((_N, 16, 16, _LQ), jnp.float32),
        mesh=mesh,
        scratch_types=[
            pltpu.VMEM((8 * _LEN,), jnp.int32),
            pltpu.VMEM((16, 4, _QC), jnp.int32),
            pltpu.VMEM((16, 4, _QC), jnp.float32),
            pltpu.VMEM((16, _QC), jnp.float32),
        ],
        compiler_params=pltpu.CompilerParams(use_tc_tiling_on_sc=False,
                                             needs_layout_passes=False,
                                             disable_bounds_checks=True),
    )(_sc_body)
    return f(val_t, widx, wwts)


def kernel(query, reference_points, input_flatten, input_spatial_shapes,
           input_level_start_index, W_off, b_off, W_attn, b_attn,
           W_val, b_val, W_out, b_out):
    del input_spatial_shapes, input_level_start_index

    woff_t = W_off.T.reshape(_M, 16, 2, _C)
    boff_t = b_off.reshape(_M, 16, 2, 1)
    widx, wwts = pl.pallas_call(
        _stage1a,
        grid=(_N, _M),
        in_specs=[
            pl.BlockSpec((1, _LQ, _C), lambda n, m: (n, 0, 0)),
            pl.BlockSpec((1, _LQ, _L, 2), lambda n, m: (n, 0, 0, 0)),
            pl.BlockSpec((1, 16, _C), lambda n, m: (m, 0, 0)),
            pl.BlockSpec((1, 16, _C), lambda n, m: (m, 0, 0)),
            pl.BlockSpec((1, 16, 1), lambda n, m: (m, 0, 0)),
            pl.BlockSpec((1, 16, 1), lambda n, m: (m, 0, 0)),
            pl.BlockSpec((1, 16, _C), lambda n, m: (m, 0, 0)),
            pl.BlockSpec((1, 16, 1), lambda n, m: (m, 0, 0)),
        ],
        out_specs=[
            pl.BlockSpec((1, 16, 4, _LQ), lambda n, m: (n, m, 0, 0)),
            pl.BlockSpec((1, 16, 4, _LQ), lambda n, m: (n, m, 0, 0)),
        ],
        out_shape=[
            jax.ShapeDtypeStruct((_N, 128, 4, _LQ), jnp.int32),
            jax.ShapeDtypeStruct((_N, 128, 4, _LQ), jnp.float32),
        ],
        compiler_params=pltpu.CompilerParams(vmem_limit_bytes=100 * 1024 * 1024),
    )(query, reference_points,
      woff_t[:, :, 0, :], woff_t[:, :, 1, :],
      boff_t[:, :, 0, :], boff_t[:, :, 1, :],
      W_attn.T.reshape(_M, 16, _C), b_attn.reshape(_M, 16, 1))

    val_t = pl.pallas_call(
        _stage1b,
        grid=(_N,),
        in_specs=[
            pl.BlockSpec((1, _LQ, _C), lambda n: (n, 0, 0)),
            pl.BlockSpec((_C, _C), lambda n: (0, 0)),
            pl.BlockSpec((1, _C), lambda n: (0, 0)),
        ],
        out_specs=pl.BlockSpec((1, 16, 8, _LEN), lambda n: (n, 0, 0, 0)),
        out_shape=jax.ShapeDtypeStruct((_N, 16, 8, _LEN), jnp.int32),
    )(input_flatten, W_val, b_val.reshape(1, -1))
    val_t = val_t.reshape(_N, 16, 8 * _LEN)

    attn = _sc_sample(val_t, widx, wwts)
    attn_r = attn.reshape(_N, _C, _LQ)

    out = pl.pallas_call(
        _stage3,
        grid=(_N,),
        in_specs=[
            pl.BlockSpec((1, _C, _LQ), lambda n: (n, 0, 0)),
            pl.BlockSpec((_C, _C), lambda n: (0, 0)),
            pl.BlockSpec((1, _C), lambda n: (0, 0)),
        ],
        out_specs=pl.BlockSpec((1, _LQ, _C), lambda n: (n, 0, 0)),
        out_shape=jax.ShapeDtypeStruct((_N, _LQ, _C), jnp.float32),
    )(attn_r, W_out, b_out.reshape(1, -1))
    return out
